# baseline (speedup 1.0000x reference)
import jax
import jax.numpy as jnp
from jax.experimental import pallas as pl
from jax.experimental.pallas import tpu as pltpu

D_MODEL = 768
HIDDEN = 768
OUT_D = 768
E = 8
TOPK = 2
T = 2048


def _moe_kernel(x_ref, Wg_ref, bg_ref, b2all_ref, W1_hbm, b1_ref, W2_hbm,
                out_ref, cvec_ref, w1buf, w2buf, sem1, sem2):
    e = pl.program_id(0)

    @pl.when(e == 0)
    def _gating():
        pltpu.make_async_copy(W1_hbm.at[0], w1buf.at[0], sem1.at[0]).start()
        pltpu.make_async_copy(W2_hbm.at[0], w2buf.at[0], sem2.at[0]).start()
        logits = jax.lax.dot_general(
            Wg_ref[...], x_ref[...], (((0,), (1,)), ((), ())),
            preferred_element_type=jnp.float32) + bg_ref[...]
        m = jnp.max(logits, axis=0, keepdims=True)
        ex = jnp.exp(logits - m)
        probs = ex / jnp.sum(ex, axis=0, keepdims=True)
        row = jax.lax.broadcasted_iota(jnp.int32, probs.shape, 0)
        m1 = jnp.max(probs, axis=0, keepdims=True)
        idx1 = jnp.min(jnp.where(probs == m1, row, E), axis=0, keepdims=True)
        mask1 = row == idx1
        probsm = jnp.where(mask1, -jnp.inf, probs)
        m2 = jnp.max(probsm, axis=0, keepdims=True)
        idx2 = jnp.min(jnp.where(probsm == m2, row, E), axis=0, keepdims=True)
        mask2 = row == idx2
        denom = m1 + m2 + 1e-9
        combine_t = jnp.where(mask1 | mask2, probs, 0.0) / denom
        cvec_ref[...] = combine_t[:, :, None]
        out_ref[...] = jax.lax.dot_general(
            combine_t, b2all_ref[...], (((0,), (0,)), ((), ())),
            preferred_element_type=jnp.float32)

    slot = jax.lax.rem(e, 2)
    nslot = jax.lax.rem(e + 1, 2)

    @pl.when(e < E - 1)
    def _prefetch():
        pltpu.make_async_copy(W1_hbm.at[e + 1], w1buf.at[nslot],
                              sem1.at[nslot]).start()
        pltpu.make_async_copy(W2_hbm.at[e + 1], w2buf.at[nslot],
                              sem2.at[nslot]).start()

    pltpu.make_async_copy(W1_hbm.at[e], w1buf.at[slot], sem1.at[slot]).wait()
    h = jax.nn.relu(jnp.dot(x_ref[...], w1buf[slot],
                            preferred_element_type=jnp.float32) + b1_ref[e])
    pltpu.make_async_copy(W2_hbm.at[e], w2buf.at[slot], sem2.at[slot]).wait()
    y = jnp.dot(h, w2buf[slot], preferred_element_type=jnp.float32)
    out_ref[...] += cvec_ref[e] * y


def kernel(x, Wg, bg, W1, b1, W2, b2):
    bg2 = bg.reshape(E, 1)
    b1r = b1.reshape(E, 1, HIDDEN)
    return pl.pallas_call(
        _moe_kernel,
        grid=(E,),
        in_specs=[
            pl.BlockSpec((T, D_MODEL), lambda e: (0, 0)),
            pl.BlockSpec((D_MODEL, E), lambda e: (0, 0)),
            pl.BlockSpec((E, 1), lambda e: (0, 0)),
            pl.BlockSpec((E, OUT_D), lambda e: (0, 0)),
            pl.BlockSpec(memory_space=pltpu.MemorySpace.HBM),
            pl.BlockSpec((E, 1, HIDDEN), lambda e: (0, 0, 0)),
            pl.BlockSpec(memory_space=pltpu.MemorySpace.HBM),
        ],
        out_specs=pl.BlockSpec((T, OUT_D), lambda e: (0, 0)),
        out_shape=jax.ShapeDtypeStruct((T, OUT_D), x.dtype),
        scratch_shapes=[
            pltpu.VMEM((E, T, 1), jnp.float32),
            pltpu.VMEM((2, D_MODEL, HIDDEN), jnp.float32),
            pltpu.VMEM((2, HIDDEN, OUT_D), jnp.float32),
            pltpu.SemaphoreType.DMA((2,)),
            pltpu.SemaphoreType.DMA((2,)),
        ],
        compiler_params=pltpu.CompilerParams(
            dimension_semantics=("arbitrary",),
        ),
    )(x, Wg, bg2, b2, W1, b1r, W2)


# speedup vs baseline: 1.1811x; 1.1811x over previous
import jax
import jax.numpy as jnp
from jax.experimental import pallas as pl
from jax.experimental.pallas import tpu as pltpu

D_MODEL = 768
HIDDEN = 768
OUT_D = 768
E = 8
TOPK = 2
T = 2048
TH = T // 2


def _moe_kernel(x_ref, Wg_ref, bg_ref, b2all_ref, W1_ref, b1_ref, W2_ref,
                out_ref, cvec_ref):
    e = pl.program_id(0)

    @pl.when(e == 0)
    def _gating():
        logits = jax.lax.dot_general(
            Wg_ref[...], x_ref[...], (((0,), (1,)), ((), ())),
            preferred_element_type=jnp.float32) + bg_ref[...]
        m = jnp.max(logits, axis=0, keepdims=True)
        ex = jnp.exp(logits - m)
        probs = ex / jnp.sum(ex, axis=0, keepdims=True)
        row = jax.lax.broadcasted_iota(jnp.int32, probs.shape, 0)
        m1 = jnp.max(probs, axis=0, keepdims=True)
        idx1 = jnp.min(jnp.where(probs == m1, row, E), axis=0, keepdims=True)
        mask1 = row == idx1
        probsm = jnp.where(mask1, -jnp.inf, probs)
        m2 = jnp.max(probsm, axis=0, keepdims=True)
        idx2 = jnp.min(jnp.where(probsm == m2, row, E), axis=0, keepdims=True)
        mask2 = row == idx2
        denom = m1 + m2 + 1e-9
        combine_t = jnp.where(mask1 | mask2, probs, 0.0) / denom
        cvec_ref[...] = combine_t[:, :, None]
        out_ref[...] = jax.lax.dot_general(
            combine_t, b2all_ref[...], (((0,), (0,)), ((), ())),
            preferred_element_type=jnp.float32)

    W1 = W1_ref[0]
    W2 = W2_ref[0]
    b1 = b1_ref[0]
    for half in range(2):
        rows = pl.ds(half * TH, TH)
        h = jax.nn.relu(jnp.dot(x_ref[rows, :], W1,
                                preferred_element_type=jnp.float32) + b1)
        y = jnp.dot(h, W2, preferred_element_type=jnp.float32)
        out_ref[rows, :] += cvec_ref[e, rows, :] * y


def kernel(x, Wg, bg, W1, b1, W2, b2):
    bg2 = bg.reshape(E, 1)
    b1r = b1.reshape(E, 1, HIDDEN)
    return pl.pallas_call(
        _moe_kernel,
        grid=(E,),
        in_specs=[
            pl.BlockSpec((T, D_MODEL), lambda e: (0, 0)),
            pl.BlockSpec((D_MODEL, E), lambda e: (0, 0)),
            pl.BlockSpec((E, 1), lambda e: (0, 0)),
            pl.BlockSpec((E, OUT_D), lambda e: (0, 0)),
            pl.BlockSpec((1, D_MODEL, HIDDEN), lambda e: (e, 0, 0)),
            pl.BlockSpec((1, 1, HIDDEN), lambda e: (e, 0, 0)),
            pl.BlockSpec((1, HIDDEN, OUT_D), lambda e: (e, 0, 0)),
        ],
        out_specs=pl.BlockSpec((T, OUT_D), lambda e: (0, 0)),
        out_shape=jax.ShapeDtypeStruct((T, OUT_D), x.dtype),
        scratch_shapes=[pltpu.VMEM((E, T, 1), jnp.float32)],
        compiler_params=pltpu.CompilerParams(
            dimension_semantics=("arbitrary",),
        ),
    )(x, Wg, bg2, b2, W1, b1r, W2)
